# baseline (speedup 1.0000x reference)
import functools

import jax
import jax.numpy as jnp
from jax import lax
from jax.experimental import pallas as pl
from jax.experimental.pallas import tpu as pltpu
from jax.experimental.pallas import tpu_sc as plsc

_B, _H, _W = 8, 2048, 2048
_N = 64
_LANES = 16


def _probe_gather(x_flat_hbm, b_hbm, y_hbm, xc_hbm, out_hbm,
                  b_v, y_v, xc_v, idx_v, val_v, sem):
    cid = lax.axis_index("c")
    sid = lax.axis_index("s")

    @pl.when(jnp.logical_and(cid == 0, sid == 0))
    def _():
        pltpu.sync_copy(b_hbm, b_v)
        pltpu.sync_copy(y_hbm, y_v)
        pltpu.sync_copy(xc_hbm, xc_v)
        for i in range(_N // _LANES):
            sl = pl.ds(i * _LANES, _LANES)
            idx_v[sl] = b_v[sl] * (_H * _W) + y_v[sl] * _W + xc_v[sl]
        pltpu.async_copy(x_flat_hbm.at[idx_v], val_v, sem).wait()
        pltpu.sync_copy(val_v, out_hbm)


def kernel(x, bidx, y, xc):
    x_flat = x.reshape(-1)
    mesh = plsc.VectorSubcoreMesh(core_axis_name="c", subcore_axis_name="s")
    run = functools.partial(
        pl.kernel,
        out_type=jax.ShapeDtypeStruct((_N,), jnp.float32),
        mesh=mesh,
        scratch_types=[
            pltpu.VMEM((_N,), jnp.int32),
            pltpu.VMEM((_N,), jnp.int32),
            pltpu.VMEM((_N,), jnp.int32),
            pltpu.VMEM((_N,), jnp.int32),
            pltpu.VMEM((_N,), jnp.float32),
            pltpu.SemaphoreType.DMA,
        ],
    )(_probe_gather)
    return run(x_flat, bidx, y, xc)


# speedup vs baseline: 5.0545x; 5.0545x over previous
import functools

import jax
import jax.numpy as jnp
from jax import lax
from jax.experimental import pallas as pl
from jax.experimental.pallas import tpu as pltpu
from jax.experimental.pallas import tpu_sc as plsc

_B, _H, _W = 8, 2048, 2048
_N = 64
_LANES = 16
_NTILES = _N // _LANES


def _probe_gather(x_hbm, b_hbm, y_hbm, xc_hbm, out_hbm,
                  b_v, y_v, xc_v, row_v, rows_v, val_v, sem):
    cid = lax.axis_index("c")
    sid = lax.axis_index("s")
    wid = sid * 2 + cid

    @pl.when(wid < _NTILES)
    def _():
        base = wid * _LANES
        pltpu.sync_copy(b_hbm.at[pl.ds(base, _LANES)], b_v)
        pltpu.sync_copy(y_hbm.at[pl.ds(base, _LANES)], y_v)
        pltpu.sync_copy(xc_hbm.at[pl.ds(base, _LANES)], xc_v)
        row_v[...] = b_v[...] * _H + y_v[...]
        pltpu.async_copy(x_hbm.at[row_v], rows_v, sem).wait()
        xcv = xc_v[...]
        lanes = lax.iota(jnp.int32, _LANES)
        acc = jnp.zeros((_LANES,), jnp.float32)
        for i in range(_LANES):
            xi = xcv[i]
            col0 = pl.multiple_of(xi & ~(_LANES - 1), _LANES)
            win = rows_v[i, pl.ds(col0, _LANES)]
            lane_b = jnp.full((_LANES,), xi - col0, jnp.int32)
            g = win.at[lane_b].get(mode="promise_in_bounds")
            acc = jnp.where(lanes == i, g, acc)
        val_v[...] = acc
        pltpu.sync_copy(val_v, out_hbm.at[pl.ds(base, _LANES)])


def kernel(x, bidx, y, xc):
    x2 = x.reshape(_B * _H, _W)
    mesh = plsc.VectorSubcoreMesh(core_axis_name="c", subcore_axis_name="s")
    run = functools.partial(
        pl.kernel,
        out_type=jax.ShapeDtypeStruct((_N,), jnp.float32),
        mesh=mesh,
        scratch_types=[
            pltpu.VMEM((_LANES,), jnp.int32),
            pltpu.VMEM((_LANES,), jnp.int32),
            pltpu.VMEM((_LANES,), jnp.int32),
            pltpu.VMEM((_LANES,), jnp.int32),
            pltpu.VMEM((_LANES, _W), jnp.float32),
            pltpu.VMEM((_LANES,), jnp.float32),
            pltpu.SemaphoreType.DMA,
        ],
    )(_probe_gather)
    return run(x2, bidx, y, xc)


# speedup vs baseline: 5.8285x; 1.1531x over previous
import functools

import jax
import jax.numpy as jnp
from jax import lax
from jax.experimental import pallas as pl
from jax.experimental.pallas import tpu as pltpu
from jax.experimental.pallas import tpu_sc as plsc

_B, _H, _W = 8, 2048, 2048
_N = 64
_LANES = 16
_NTILES = _N // _LANES


def _probe_gather(x_hbm, b_hbm, y_hbm, xc_hbm, out_hbm,
                  b_v, y_v, xc_v, wins_v, val_v, sem):
    sid = lax.axis_index("s")

    @pl.when(sid < _NTILES)
    def _():
        base = sid * _LANES
        pltpu.sync_copy(b_hbm.at[pl.ds(base, _LANES)], b_v)
        pltpu.sync_copy(y_hbm.at[pl.ds(base, _LANES)], y_v)
        pltpu.sync_copy(xc_hbm.at[pl.ds(base, _LANES)], xc_v)
        rows = b_v[...] * _H + y_v[...]
        xcv = xc_v[...]
        col0s = xcv & ~(_LANES - 1)
        lanes = xcv & (_LANES - 1)
        copies = []
        for i in range(_LANES):
            c0 = pl.multiple_of(col0s[i], _LANES)
            copies.append(pltpu.make_async_copy(
                x_hbm.at[rows[i], pl.ds(c0, _LANES)],
                wins_v.at[pl.ds(i * _LANES, _LANES)], sem))
        for c in copies:
            c.start()
        for c in copies:
            c.wait()
        lidx = lax.iota(jnp.int32, _LANES)
        acc = jnp.zeros((_LANES,), jnp.float32)
        for i in range(_LANES):
            win = wins_v[pl.ds(i * _LANES, _LANES)]
            lane_b = jnp.full((_LANES,), lanes[i], jnp.int32)
            g = win.at[lane_b].get(mode="promise_in_bounds")
            acc = jnp.where(lidx == i, g, acc)
        val_v[...] = acc
        pltpu.sync_copy(val_v, out_hbm.at[pl.ds(base, _LANES)])


def kernel(x, bidx, y, xc):
    x2 = x.reshape(_B * _H, _W)
    mesh = plsc.VectorSubcoreMesh(
        core_axis_name="c", subcore_axis_name="s", num_cores=1)
    run = functools.partial(
        pl.kernel,
        out_type=jax.ShapeDtypeStruct((_N,), jnp.float32),
        mesh=mesh,
        scratch_types=[
            pltpu.VMEM((_LANES,), jnp.int32),
            pltpu.VMEM((_LANES,), jnp.int32),
            pltpu.VMEM((_LANES,), jnp.int32),
            pltpu.VMEM((_LANES * _LANES,), jnp.float32),
            pltpu.VMEM((_LANES,), jnp.float32),
            pltpu.SemaphoreType.DMA,
        ],
    )(_probe_gather)
    return run(x2, bidx, y, xc)


# speedup vs baseline: 44.3726x; 7.6131x over previous
import jax
import jax.numpy as jnp
from jax import lax
from jax.experimental import pallas as pl
from jax.experimental.pallas import tpu as pltpu

_B, _H, _W = 8, 2048, 2048
_N = 64
_WIN = 128


def _probe_gather(b_s, y_s, xc_s, xc_v, x_hbm, out_ref, buf, sem):
    copies = []
    for i in range(_N):
        c0 = pl.multiple_of(xc_s[i] & ~(_WIN - 1), _WIN)
        copies.append(pltpu.make_async_copy(
            x_hbm.at[b_s[i], y_s[i], pl.ds(c0, _WIN)],
            buf.at[i], sem))
    for c in copies:
        c.start()
    for c in copies:
        c.wait()
    lane = (xc_v[...] & (_WIN - 1)).reshape(_N, 1)
    cols = lax.broadcasted_iota(jnp.int32, (_N, _WIN), 1)
    picked = jnp.where(cols == lane, buf[...], 0.0)
    out_ref[...] = jnp.sum(picked, axis=1)


def kernel(x, bidx, y, xc):
    return pl.pallas_call(
        _probe_gather,
        out_shape=jax.ShapeDtypeStruct((_N,), jnp.float32),
        in_specs=[
            pl.BlockSpec(memory_space=pltpu.SMEM),
            pl.BlockSpec(memory_space=pltpu.SMEM),
            pl.BlockSpec(memory_space=pltpu.SMEM),
            pl.BlockSpec(memory_space=pltpu.VMEM),
            pl.BlockSpec(memory_space=pl.ANY),
        ],
        out_specs=pl.BlockSpec(memory_space=pltpu.VMEM),
        scratch_shapes=[
            pltpu.VMEM((_N, _WIN), jnp.float32),
            pltpu.SemaphoreType.DMA,
        ],
    )(bidx, y, xc, xc, x)
